# Initial kernel scaffold; baseline (speedup 1.0000x reference)
import jax
import jax.numpy as jnp
from jax.experimental import pallas as pl


def kernel(X, edge_index, edge_vals, W):
    raise NotImplementedError("write your pallas kernel here")


# speedup vs baseline: 4.1798x; 4.1798x over previous
import functools

import jax
import jax.numpy as jnp
from jax import lax
from jax.experimental import pallas as pl
from jax.experimental.pallas import tpu as pltpu
from jax.experimental.pallas import tpu_sc as plsc

N = 10000
E = 320000
D = 128

NC = 2
NS = 16
NW = NC * NS
EPW = E // NW
CH = 80
NCH = EPW // CH
BLK = 80
NBLK = N // BLK


def _sc_partials(src, dst, ev, x):
    mesh = plsc.VectorSubcoreMesh(core_axis_name="c", subcore_axis_name="s")

    @functools.partial(
        pl.kernel,
        out_type=jax.ShapeDtypeStruct((NC, N, D), jnp.float32),
        mesh=mesh,
        scratch_types=[
            pltpu.VMEM((CH,), jnp.int32),
            pltpu.VMEM((CH,), jnp.int32),
            pltpu.VMEM((CH,), jnp.float32),
            pltpu.VMEM((CH, D), jnp.float32),
            pltpu.VMEM((BLK, D), jnp.float32),
            pltpu.VMEM_SHARED((N, D), jnp.float32),
            pltpu.SemaphoreType.DMA,
        ],
    )
    def body(src_hbm, dst_hbm, ev_hbm, x_hbm, out_hbm,
             src_v, dst_v, ev_v, rows_v, zbuf, acc, sem):
        c = lax.axis_index("c")
        s = lax.axis_index("s")

        zero16 = jnp.zeros((16,), jnp.float32)

        def zrow(i, carry):
            for d8 in range(D // 16):
                zbuf[i, pl.ds(d8 * 16, 16)] = zero16
            return carry

        lax.fori_loop(0, BLK, zrow, 0)
        for k in range((NBLK + NS - 1) // NS):
            b = s + k * NS

            @pl.when(b < NBLK)
            def _zcopy(b=b):
                pltpu.sync_copy(zbuf, acc.at[pl.ds(b * BLK, BLK)])

        plsc.subcore_barrier()

        ebase = (c * NS + s) * EPW

        def chunk(i, carry):
            base = ebase + i * CH
            pltpu.sync_copy(src_hbm.at[pl.ds(base, CH)], src_v)
            pltpu.sync_copy(dst_hbm.at[pl.ds(base, CH)], dst_v)
            pltpu.sync_copy(ev_hbm.at[pl.ds(base, CH)], ev_v)
            pltpu.async_copy(x_hbm.at[src_v], rows_v, sem).wait()

            def scale(g, c2):
                ev16 = ev_v[pl.ds(g * 16, 16)]
                for j in range(16):
                    v = ev16[j]
                    e = g * 16 + j
                    for d8 in range(D // 16):
                        sl = pl.ds(d8 * 16, 16)
                        rows_v[e, sl] = rows_v[e, sl] * v
                return c2

            lax.fori_loop(0, CH // 16, scale, 0)
            pltpu.sync_copy(rows_v, acc.at[dst_v], add=True)
            return carry

        lax.fori_loop(0, NCH, chunk, 0)
        plsc.subcore_barrier()

        for k in range((NBLK + NS - 1) // NS):
            b = s + k * NS

            @pl.when(b < NBLK)
            def _ocopy(b=b):
                r0 = b * BLK
                pltpu.sync_copy(acc.at[pl.ds(r0, BLK)],
                                out_hbm.at[c, pl.ds(r0, BLK)])

    return body(src, dst, ev, x)


BM = 1000


def _tc_body(p_ref, wt_ref, o_ref):
    ax = p_ref[0] + p_ref[1]
    h = jnp.dot(ax, wt_ref[...], preferred_element_type=jnp.float32)
    o_ref[...] = jnp.maximum(h, 0.0)


def _tc_linrelu(partials, wt):
    return pl.pallas_call(
        _tc_body,
        grid=(N // BM,),
        in_specs=[
            pl.BlockSpec((NC, BM, D), lambda i: (0, i, 0)),
            pl.BlockSpec((D, D), lambda i: (0, 0)),
        ],
        out_specs=pl.BlockSpec((BM, D), lambda i: (i, 0)),
        out_shape=jax.ShapeDtypeStruct((N, D), jnp.float32),
    )(partials, wt)


def kernel(X, edge_index, edge_vals, W):
    dst = edge_index[0]
    src = edge_index[1]
    partials = _sc_partials(src, dst, edge_vals, X)
    return _tc_linrelu(partials, W.T)


# speedup vs baseline: 10.4246x; 2.4940x over previous
import functools

import jax
import jax.numpy as jnp
from jax import lax
from jax.experimental import pallas as pl
from jax.experimental.pallas import tpu as pltpu
from jax.experimental.pallas import tpu_sc as plsc

N = 10000
E = 320000
D = 128

NC = 2
NS = 16
NW = NC * NS
EPW = E // NW
CH = 80
NCH = EPW // CH
NBUF = 3
GS = 25
BLK = 80
NBLK = N // BLK


def _sc_partials(src, dst, ev, x):
    mesh = plsc.VectorSubcoreMesh(core_axis_name="c", subcore_axis_name="s")

    @functools.partial(
        pl.kernel,
        out_type=jax.ShapeDtypeStruct((NC, N, D), jnp.float32),
        mesh=mesh,
        scratch_types=[
            pltpu.VMEM((GS, CH), jnp.int32),
            pltpu.VMEM((GS, CH), jnp.float32),
            pltpu.VMEM((2, GS, CH), jnp.int32),
            [pltpu.VMEM((CH, D), jnp.float32) for _ in range(NBUF)],
            pltpu.VMEM_SHARED((N, D), jnp.float32),
            [pltpu.SemaphoreType.DMA for _ in range(NBUF)],
            [pltpu.SemaphoreType.DMA for _ in range(NBUF)],
        ],
    )
    def body(src_hbm, dst_hbm, ev_hbm, x_hbm, out_hbm,
             src_sl, ev_sl, dst_sl, rows, acc, gsem, ssem):
        c = lax.axis_index("c")
        s = lax.axis_index("s")
        w = c * NS + s

        zero16 = jnp.zeros((16,), jnp.float32)

        def zrow(i, carry):
            for d8 in range(D // 16):
                rows[0][i, pl.ds(d8 * 16, 16)] = zero16
            return carry

        lax.fori_loop(0, CH, zrow, 0)
        for k in range((NBLK + NS - 1) // NS):
            blk = s + k * NS

            @pl.when(blk < NBLK)
            def _zcopy(blk=blk):
                pltpu.sync_copy(rows[0], acc.at[pl.ds(blk * BLK, BLK)])

        plsc.subcore_barrier()

        def step(i, b):
            nb = (b + 1) % NBUF
            local = lax.rem(i, GS)
            g = lax.div(i, GS)
            p = lax.rem(g, 2)

            @pl.when(local == 0)
            def _slab():
                pltpu.sync_copy(src_hbm.at[w, g], src_sl)
                pltpu.sync_copy(ev_hbm.at[w, g], ev_sl)
                pltpu.sync_copy(dst_hbm.at[w, g], dst_sl.at[p])
                pltpu.async_copy(x_hbm.at[src_sl.at[0]], rows[b], gsem[b])

            @pl.when(i >= 2)
            def _wait_sct():
                pltpu.make_async_copy(
                    rows[nb], acc.at[dst_sl.at[0, 0]], ssem[nb]).wait()

            @pl.when(lax.rem(i + 1, GS) != 0)
            def _prefetch():
                pltpu.async_copy(
                    x_hbm.at[src_sl.at[local + 1]], rows[nb], gsem[nb])

            pltpu.make_async_copy(
                x_hbm.at[src_sl.at[local]], rows[b], gsem[b]).wait()

            def scale(g, c2):
                ev16 = ev_sl[local, pl.ds(g * 16, 16)]
                for j16 in range(16):
                    v = ev16[j16]
                    e = g * 16 + j16
                    for d8 in range(D // 16):
                        sl = pl.ds(d8 * 16, 16)
                        rows[b][e, sl] = rows[b][e, sl] * v
                return c2

            lax.fori_loop(0, CH // 16, scale, 0)
            pltpu.async_copy(
                rows[b], acc.at[dst_sl.at[p, local]], ssem[b], add=True)

        def group(j, carry):
            for t in range(NBUF):
                step(j * NBUF + t, t)
            return carry

        lax.fori_loop(0, (NCH - 2) // NBUF, group, 0)
        step(jnp.int32(NCH - 2), 0)
        step(jnp.int32(NCH - 1), 1)
        for b in (0, 1):
            pltpu.make_async_copy(
                rows[b], acc.at[dst_sl.at[0, 0]], ssem[b]).wait()
        plsc.subcore_barrier()

        for k in range((NBLK + NS - 1) // NS):
            blk = s + k * NS

            @pl.when(blk < NBLK)
            def _ocopy(blk=blk):
                r0 = blk * BLK
                pltpu.sync_copy(acc.at[pl.ds(r0, BLK)],
                                out_hbm.at[c, pl.ds(r0, BLK)])

    return body(src, dst, ev, x)


BM = 1000


def _tc_body(p_ref, wt_ref, o_ref):
    ax = p_ref[0] + p_ref[1]
    h = jnp.dot(ax, wt_ref[...], preferred_element_type=jnp.float32)
    o_ref[...] = jnp.maximum(h, 0.0)


def _tc_linrelu(partials, wt):
    return pl.pallas_call(
        _tc_body,
        grid=(N // BM,),
        in_specs=[
            pl.BlockSpec((NC, BM, D), lambda i: (0, i, 0)),
            pl.BlockSpec((D, D), lambda i: (0, 0)),
        ],
        out_specs=pl.BlockSpec((BM, D), lambda i: (i, 0)),
        out_shape=jax.ShapeDtypeStruct((N, D), jnp.float32),
    )(partials, wt)


def kernel(X, edge_index, edge_vals, W):
    dst = edge_index[0].reshape(NW, NCH // GS, GS, CH)
    src = edge_index[1].reshape(NW, NCH // GS, GS, CH)
    ev = edge_vals.reshape(NW, NCH // GS, GS, CH)
    partials = _sc_partials(src, dst, ev, X)
    return _tc_linrelu(partials, W.T)
